# baseline (speedup 1.0000x reference)
import jax
import jax.numpy as jnp
from jax.experimental import pallas as pl
from jax.experimental.pallas import tpu as pltpu


_BM = 400
_NPANEL = 24
_CR = 80
_NCHUNK = 5
_NSLOT = 3


def _gcn_kernel(x_ref, adj_win_ref, w_hbm_ref, b_hbm_ref, adj_hbm_ref,
                out_ref, h_ref, w_ref, b_ref, tail_ref, sem_wb_ref, sem_ref):
    i = pl.program_id(0)
    base = _NPANEL * _BM

    @pl.when(i == 0)
    def _compute_h():
        cw = pltpu.make_async_copy(w_hbm_ref, w_ref, sem_wb_ref.at[0])
        cb = pltpu.make_async_copy(b_hbm_ref, b_ref, sem_wb_ref.at[1])
        cw.start()
        cb.start()
        cw.wait()
        cb.wait()
        h_ref[...] = jnp.dot(
            x_ref[...], w_ref[...], preferred_element_type=jnp.float32
        )

    @pl.when(i == _NPANEL - 1)
    def _issue_tail():
        for s in range(_NSLOT):
            pltpu.make_async_copy(
                adj_hbm_ref.at[pl.ds(base + s * _CR, _CR), :],
                tail_ref.at[s],
                sem_ref.at[s],
            ).start()

    @pl.when(i < _NPANEL)
    def _main():
        out_ref[...] = (
            jnp.dot(adj_win_ref[...], h_ref[...],
                    preferred_element_type=jnp.float32)
            + b_ref[...]
        )

    @pl.when(i == _NPANEL)
    def _tail():
        def body(k, carry):
            slot = jax.lax.rem(k, _NSLOT)
            pltpu.make_async_copy(
                adj_hbm_ref.at[pl.ds(base + k * _CR, _CR), :],
                tail_ref.at[slot],
                sem_ref.at[slot],
            ).wait()
            out_ref[pl.ds(k * _CR, _CR), :] = (
                jnp.dot(tail_ref[slot], h_ref[...],
                        preferred_element_type=jnp.float32)
                + b_ref[...]
            )

            @pl.when(k + _NSLOT < _NCHUNK)
            def _issue_next():
                pltpu.make_async_copy(
                    adj_hbm_ref.at[pl.ds(base + (k + _NSLOT) * _CR, _CR), :],
                    tail_ref.at[slot],
                    sem_ref.at[slot],
                ).start()

            return carry

        jax.lax.fori_loop(0, _NCHUNK, body, 0)


def kernel(x, adj, W, b):
    n, d_in = x.shape
    d_out = W.shape[1]
    out = pl.pallas_call(
        _gcn_kernel,
        grid=(_NPANEL + 1,),
        in_specs=[
            pl.BlockSpec((n, d_in), lambda i: (0, 0)),
            pl.BlockSpec((_BM, n), lambda i: (jnp.minimum(i, _NPANEL - 1), 0)),
            pl.BlockSpec(memory_space=pltpu.MemorySpace.HBM),
            pl.BlockSpec(memory_space=pltpu.MemorySpace.HBM),
            pl.BlockSpec(memory_space=pltpu.MemorySpace.HBM),
        ],
        out_specs=pl.BlockSpec((_BM, d_out), lambda i: (i, 0)),
        out_shape=jax.ShapeDtypeStruct((n, d_out), jnp.float32),
        scratch_shapes=[
            pltpu.VMEM((n, d_out), jnp.float32),
            pltpu.VMEM((d_in, d_out), jnp.float32),
            pltpu.VMEM((1, d_out), jnp.float32),
            pltpu.VMEM((_NSLOT, _CR, n), jnp.float32),
            pltpu.SemaphoreType.DMA((2,)),
            pltpu.SemaphoreType.DMA((_NSLOT,)),
        ],
        compiler_params=pltpu.CompilerParams(
            vmem_limit_bytes=64 * 1024 * 1024,
        ),
    )(x, adj, W, b.reshape(1, d_out), adj)
    return out.reshape(1, n, d_out)


# speedup vs baseline: 1.0346x; 1.0346x over previous
import jax
import jax.numpy as jnp
from jax.experimental import pallas as pl
from jax.experimental.pallas import tpu as pltpu


_BM = 400
_NPANEL = 24
_CR = 80
_NCHUNK = 5
_NSLOT = 3


def _gcn_kernel(x_ref, w_ref, b_ref, adj_win_ref, adj_hbm_ref, out_ref,
                h_ref, tail_ref, sem_ref):
    i = pl.program_id(0)
    base = _NPANEL * _BM

    @pl.when(i == 0)
    def _compute_h():
        h_ref[...] = jnp.dot(
            x_ref[...], w_ref[...], preferred_element_type=jnp.float32
        )

    @pl.when(i == _NPANEL - 1)
    def _issue_tail():
        for s in range(_NSLOT):
            pltpu.make_async_copy(
                adj_hbm_ref.at[pl.ds(base + s * _CR, _CR), :],
                tail_ref.at[s],
                sem_ref.at[s],
            ).start()

    @pl.when(i < _NPANEL)
    def _main():
        out_ref[...] = (
            jnp.dot(adj_win_ref[...], h_ref[...],
                    preferred_element_type=jnp.float32)
            + b_ref[...]
        )

    @pl.when(i == _NPANEL)
    def _tail():
        def body(k, carry):
            slot = jax.lax.rem(k, _NSLOT)
            pltpu.make_async_copy(
                adj_hbm_ref.at[pl.ds(base + k * _CR, _CR), :],
                tail_ref.at[slot],
                sem_ref.at[slot],
            ).wait()
            out_ref[pl.ds(k * _CR, _CR), :] = (
                jnp.dot(tail_ref[slot], h_ref[...],
                        preferred_element_type=jnp.float32)
                + b_ref[...]
            )

            @pl.when(k + _NSLOT < _NCHUNK)
            def _issue_next():
                pltpu.make_async_copy(
                    adj_hbm_ref.at[pl.ds(base + (k + _NSLOT) * _CR, _CR), :],
                    tail_ref.at[slot],
                    sem_ref.at[slot],
                ).start()

            return carry

        jax.lax.fori_loop(0, _NCHUNK, body, 0)


def kernel(x, adj, W, b):
    n, d_in = x.shape
    d_out = W.shape[1]
    out = pl.pallas_call(
        _gcn_kernel,
        grid=(_NPANEL + 1,),
        in_specs=[
            pl.BlockSpec((n, d_in), lambda i: (0, 0)),
            pl.BlockSpec((d_in, d_out), lambda i: (0, 0)),
            pl.BlockSpec((1, d_out), lambda i: (0, 0)),
            pl.BlockSpec((_BM, n), lambda i: (jnp.minimum(i, _NPANEL - 1), 0)),
            pl.BlockSpec(memory_space=pltpu.MemorySpace.HBM),
        ],
        out_specs=pl.BlockSpec((_BM, d_out), lambda i: (i, 0)),
        out_shape=jax.ShapeDtypeStruct((n, d_out), jnp.float32),
        scratch_shapes=[
            pltpu.VMEM((n, d_out), jnp.float32),
            pltpu.VMEM((_NSLOT, _CR, n), jnp.float32),
            pltpu.SemaphoreType.DMA((_NSLOT,)),
        ],
        compiler_params=pltpu.CompilerParams(
            vmem_limit_bytes=64 * 1024 * 1024,
        ),
    )(x, W, b.reshape(1, d_out), adj, adj)
    return out.reshape(1, n, d_out)
